# baseline (speedup 1.0000x reference)
import functools

import jax
import jax.numpy as jnp
from jax import lax
from jax.experimental import pallas as pl
from jax.experimental.pallas import tpu as pltpu
from jax.experimental.pallas import tpu_sc as plsc

_NC, _NS, _L = 2, 16, 16
_NW = _NC * _NS
_TAB_PAD = 352
_SB = 8
_BLK = 128
_NB = 4


@functools.lru_cache(maxsize=None)
def _build(nbatch, seqlen):
    nblk_b = nbatch // _BLK
    nblk_s = seqlen // _SB
    chunks_s = nblk_b // _NB
    n_chunks = nblk_s * chunks_s
    per_w = n_chunks // _NW
    assert per_w * _NW == n_chunks and per_w >= 3 and per_w % 2 == 1
    assert nblk_b * _BLK == nbatch and nblk_s * _SB == seqlen
    mesh = plsc.VectorSubcoreMesh(core_axis_name="c", subcore_axis_name="s")

    @functools.partial(
        pl.kernel,
        out_type=jax.ShapeDtypeStruct((seqlen, nblk_b, 4, _BLK), jnp.float32),
        mesh=mesh,
        scratch_types=[
            pltpu.VMEM((_TAB_PAD,), jnp.float32),
            pltpu.VMEM((3, _NB, _SB, _BLK), jnp.int32),
            pltpu.VMEM((3, _SB, _NB, 4, _BLK), jnp.float32),
            pltpu.SemaphoreType.DMA,
            pltpu.SemaphoreType.DMA,
            pltpu.SemaphoreType.DMA,
            pltpu.SemaphoreType.DMA,
            pltpu.SemaphoreType.DMA,
            pltpu.SemaphoreType.DMA,
        ],
        compiler_params=pltpu.CompilerParams(
            needs_layout_passes=False, use_tc_tiling_on_sc=False),
    )
    def k(ids_hbm, tab_hbm, out_hbm, tab_v, ids_v, out_v,
          isem0, isem1, isem2, osem0, osem1, osem2):
        isems = (isem0, isem1, isem2)
        osems = (osem0, osem1, osem2)
        wid = lax.axis_index("s") * _NC + lax.axis_index("c")
        k0 = wid * per_w
        pltpu.sync_copy(tab_hbm, tab_v)

        def ids_src(t):
            kk = k0 + t
            return ids_hbm.at[kk // chunks_s, pl.ds((kk % chunks_s) * _NB, _NB)]

        def out_dst(t):
            kk = k0 + t
            return out_hbm.at[pl.ds((kk // chunks_s) * _SB, _SB),
                              pl.ds((kk % chunks_s) * _NB, _NB)]

        def start_ids(t, buf):
            pltpu.async_copy(ids_src(t), ids_v.at[buf], isems[buf])

        def wait_ids(t, buf):
            pltpu.make_async_copy(ids_src(t), ids_v.at[buf],
                                  isems[buf]).wait()

        def start_out(t, buf):
            pltpu.async_copy(out_v.at[buf], out_dst(t), osems[buf])

        def wait_out(t, buf):
            pltpu.make_async_copy(out_v.at[buf], out_dst(t),
                                  osems[buf]).wait()

        def compute(buf):
            @plsc.parallel_loop(0, _NB * _SB, unroll=1)
            def q_body(q):
                bb = q // _SB
                s_lo = q % _SB
                for g in range(_BLK // _L):
                    ids16 = ids_v[buf, bb, s_lo, pl.ds(g * _L, _L)]
                    ids4 = ids16 * 4
                    for c in range(4):
                        vals = plsc.load_gather(tab_v, [ids4 + c])
                        out_v[buf, s_lo, bb, c, pl.ds(g * _L, _L)] = vals

        NBUF = 3
        n_loop = (per_w - NBUF - 1) // NBUF
        assert per_w == NBUF + NBUF * n_loop + 1

        def phase(t, buf, first=False, static=False):
            if not first:
                wait_out(t - NBUF, buf)
            wait_ids(t, buf)
            compute(buf)
            start_out(t, buf)
            nxt = t + NBUF
            if static:
                if nxt < per_w:
                    start_ids(nxt, buf)
            else:
                @pl.when(nxt < per_w)
                def _():
                    start_ids(nxt, buf)

        for b in range(NBUF):
            start_ids(b, b)
        for b in range(NBUF):
            phase(b, b, first=True, static=True)

        def loop_body(i, carry):
            for b in range(NBUF):
                phase(NBUF * i + b, b)
            return carry

        lax.fori_loop(1, 1 + n_loop, loop_body, 0)
        t_last = per_w - 1
        phase(t_last, t_last % NBUF, static=True)
        for t in range(per_w - NBUF, per_w):
            wait_out(t, t % NBUF)

    return k


def kernel(dna, embedding_table):
    nbatch, seqlen = dna.shape
    tab = jnp.pad(embedding_table.reshape(-1),
                  (0, _TAB_PAD - embedding_table.size))
    ids_native = (dna.T.reshape(seqlen // _SB, _SB, nbatch // _BLK, _BLK)
                  .transpose(0, 2, 1, 3))
    y = _build(nbatch, seqlen)(ids_native, tab)
    return y.transpose(1, 3, 0, 2).reshape(nbatch, seqlen, 1, 4)


# speedup vs baseline: 1.0662x; 1.0662x over previous
import functools

import jax
import jax.numpy as jnp
from jax import lax
from jax.experimental import pallas as pl
from jax.experimental.pallas import tpu as pltpu
from jax.experimental.pallas import tpu_sc as plsc

_NC, _NS, _L = 2, 16, 16
_NW = _NC * _NS
_TAB_PAD = 352
_SB = 8
_BLK = 128
_NB = 4


@functools.lru_cache(maxsize=None)
def _build(nbatch, seqlen):
    nblk_b = nbatch // _BLK
    nblk_s = seqlen // _SB
    chunks_s = nblk_b // _NB
    n_chunks = nblk_s * chunks_s
    per_w = n_chunks // _NW
    assert per_w * _NW == n_chunks and per_w >= 3 and per_w % 2 == 1
    assert nblk_b * _BLK == nbatch and nblk_s * _SB == seqlen
    mesh = plsc.VectorSubcoreMesh(core_axis_name="c", subcore_axis_name="s")

    @functools.partial(
        pl.kernel,
        out_type=jax.ShapeDtypeStruct((seqlen, nblk_b, 4, _BLK), jnp.float32),
        mesh=mesh,
        scratch_types=[
            pltpu.VMEM((_TAB_PAD,), jnp.float32),
            pltpu.VMEM((2, _NB, _SB, _BLK), jnp.int32),
            pltpu.VMEM((2, _SB, _NB, 4, _BLK), jnp.float32),
            pltpu.SemaphoreType.DMA,
            pltpu.SemaphoreType.DMA,
            pltpu.SemaphoreType.DMA,
            pltpu.SemaphoreType.DMA,
        ],
        compiler_params=pltpu.CompilerParams(
            needs_layout_passes=False, use_tc_tiling_on_sc=False),
    )
    def k(ids_hbm, tab_hbm, out_hbm, tab_v, ids_v, out_v,
          isem0, isem1, osem0, osem1):
        isems = (isem0, isem1)
        osems = (osem0, osem1)
        wid = lax.axis_index("s") * _NC + lax.axis_index("c")
        k0 = wid * per_w
        pltpu.sync_copy(tab_hbm, tab_v)

        def ids_src(t):
            kk = k0 + t
            return ids_hbm.at[kk // chunks_s, pl.ds((kk % chunks_s) * _NB, _NB)]

        def out_dst(t):
            kk = k0 + t
            return out_hbm.at[pl.ds((kk // chunks_s) * _SB, _SB),
                              pl.ds((kk % chunks_s) * _NB, _NB)]

        def start_ids(t, buf):
            pltpu.async_copy(ids_src(t), ids_v.at[buf], isems[buf])

        def wait_ids(t, buf):
            pltpu.make_async_copy(ids_src(t), ids_v.at[buf],
                                  isems[buf]).wait()

        def start_out(t, buf):
            pltpu.async_copy(out_v.at[buf], out_dst(t), osems[buf])

        def wait_out(t, buf):
            pltpu.make_async_copy(out_v.at[buf], out_dst(t),
                                  osems[buf]).wait()

        def compute(buf):
            @plsc.parallel_loop(0, _NB * _SB, unroll=1)
            def q_body(q):
                bb = q // _SB
                s_lo = q % _SB
                for g in range(_BLK // _L):
                    ids16 = ids_v[buf, bb, s_lo, pl.ds(g * _L, _L)]
                    ids4 = ids16 * 4
                    for c in range(4):
                        vals = plsc.load_gather(tab_v, [ids4 + c])
                        out_v[buf, s_lo, bb, c, pl.ds(g * _L, _L)] = vals

        start_ids(0, 0)
        start_ids(1, 1)
        wait_ids(0, 0)
        compute(0)
        start_out(0, 0)
        start_ids(2, 0)
        wait_ids(1, 1)
        compute(1)
        start_out(1, 1)
        start_ids(3, 1)

        def loop_body(i, carry):
            t_even = 2 * i
            wait_out(t_even - 2, 0)
            wait_ids(t_even, 0)
            compute(0)
            start_out(t_even, 0)
            start_ids(t_even + 2, 0)
            t_odd = 2 * i + 1
            wait_out(t_odd - 2, 1)
            wait_ids(t_odd, 1)
            compute(1)
            start_out(t_odd, 1)

            @pl.when(t_odd + 2 < per_w)
            def _():
                start_ids(t_odd + 2, 1)
            return carry

        lax.fori_loop(1, (per_w - 1) // 2, loop_body, 0)
        t_last = per_w - 1
        wait_out(t_last - 2, 0)
        wait_ids(t_last, 0)
        compute(0)
        start_out(t_last, 0)
        wait_out(t_last - 1, 1)
        wait_out(t_last, 0)

    return k


def kernel(dna, embedding_table):
    nbatch, seqlen = dna.shape
    tab = jnp.pad(embedding_table.reshape(-1),
                  (0, _TAB_PAD - embedding_table.size))
    ids_native = (dna.T.reshape(seqlen // _SB, _SB, nbatch // _BLK, _BLK)
                  .transpose(0, 2, 1, 3))
    y = _build(nbatch, seqlen)(ids_native, tab)
    return y.transpose(1, 3, 0, 2).reshape(nbatch, seqlen, 1, 4)


# speedup vs baseline: 1.2654x; 1.1869x over previous
import functools

import jax
import jax.numpy as jnp
from jax import lax
from jax.experimental import pallas as pl
from jax.experimental.pallas import tpu as pltpu
from jax.experimental.pallas import tpu_sc as plsc

_NC, _NS, _L = 2, 16, 16
_NW = _NC * _NS
_TAB_PAD = 352
_SB = 8
_BLK = 128
_NB = 4


@functools.lru_cache(maxsize=None)
def _build(nbatch, seqlen):
    nblk_b = nbatch // _BLK
    nblk_s = seqlen // _SB
    chunks_s = nblk_b // _NB
    n_chunks = nblk_s * chunks_s
    per_w = n_chunks // _NW
    assert per_w * _NW == n_chunks and per_w >= 3 and per_w % 2 == 1
    assert nblk_b * _BLK == nbatch and nblk_s * _SB == seqlen
    mesh = plsc.VectorSubcoreMesh(core_axis_name="c", subcore_axis_name="s")

    @functools.partial(
        pl.kernel,
        out_type=jax.ShapeDtypeStruct((seqlen, nblk_b, 4, _BLK), jnp.float32),
        mesh=mesh,
        scratch_types=[
            pltpu.VMEM((_TAB_PAD,), jnp.int32),
            pltpu.VMEM((2, _NB, _SB, _BLK), jnp.int32),
            pltpu.VMEM((2, _SB, _NB, 4, _BLK), jnp.float32),
            pltpu.SemaphoreType.DMA,
            pltpu.SemaphoreType.DMA,
            pltpu.SemaphoreType.DMA,
            pltpu.SemaphoreType.DMA,
        ],
        compiler_params=pltpu.CompilerParams(
            needs_layout_passes=False, use_tc_tiling_on_sc=False),
    )
    def k(ids_hbm, tab_hbm, out_hbm, tab_v, ids_v, out_v,
          isem0, isem1, osem0, osem1):
        isems = (isem0, isem1)
        osems = (osem0, osem1)
        wid = lax.axis_index("s") * _NC + lax.axis_index("c")
        k0 = wid * per_w
        pltpu.sync_copy(tab_hbm, tab_v)

        def ids_src(t):
            kk = k0 + t
            return ids_hbm.at[kk // chunks_s, pl.ds((kk % chunks_s) * _NB, _NB)]

        def out_dst(t):
            kk = k0 + t
            return out_hbm.at[pl.ds((kk // chunks_s) * _SB, _SB),
                              pl.ds((kk % chunks_s) * _NB, _NB)]

        def start_ids(t, buf):
            pltpu.async_copy(ids_src(t), ids_v.at[buf], isems[buf])

        def wait_ids(t, buf):
            pltpu.make_async_copy(ids_src(t), ids_v.at[buf],
                                  isems[buf]).wait()

        def start_out(t, buf):
            pltpu.async_copy(out_v.at[buf], out_dst(t), osems[buf])

        def wait_out(t, buf):
            pltpu.make_async_copy(out_v.at[buf], out_dst(t),
                                  osems[buf]).wait()

        himask = jnp.full((_L,), -65536, jnp.int32)

        def compute(buf):
            @plsc.parallel_loop(0, _NB * _SB, unroll=1)
            def q_body(q):
                bb = q // _SB
                s_lo = q % _SB
                for g in range(_BLK // _L):
                    ids16 = ids_v[buf, bb, s_lo, pl.ds(g * _L, _L)]
                    ids2 = ids16 * 2
                    for h in range(2):
                        w = plsc.load_gather(tab_v, [ids2 + h])
                        lo = plsc.bitcast(w << 16, jnp.float32)
                        hi = plsc.bitcast(w & himask, jnp.float32)
                        out_v[buf, s_lo, bb, 2 * h, pl.ds(g * _L, _L)] = lo
                        out_v[buf, s_lo, bb, 2 * h + 1, pl.ds(g * _L, _L)] = hi

        start_ids(0, 0)
        start_ids(1, 1)
        wait_ids(0, 0)
        compute(0)
        start_out(0, 0)
        start_ids(2, 0)
        wait_ids(1, 1)
        compute(1)
        start_out(1, 1)
        start_ids(3, 1)

        def loop_body(i, carry):
            t_even = 2 * i
            wait_out(t_even - 2, 0)
            wait_ids(t_even, 0)
            compute(0)
            start_out(t_even, 0)
            start_ids(t_even + 2, 0)
            t_odd = 2 * i + 1
            wait_out(t_odd - 2, 1)
            wait_ids(t_odd, 1)
            compute(1)
            start_out(t_odd, 1)

            @pl.when(t_odd + 2 < per_w)
            def _():
                start_ids(t_odd + 2, 1)
            return carry

        lax.fori_loop(1, (per_w - 1) // 2, loop_body, 0)
        t_last = per_w - 1
        wait_out(t_last - 2, 0)
        wait_ids(t_last, 0)
        compute(0)
        start_out(t_last, 0)
        wait_out(t_last - 1, 1)
        wait_out(t_last, 0)

    return k


def kernel(dna, embedding_table):
    nbatch, seqlen = dna.shape
    u16 = lax.bitcast_convert_type(
        embedding_table.astype(jnp.bfloat16), jnp.uint16).astype(jnp.uint32)
    packed = (u16[:, 0::2] | (u16[:, 1::2] << 16)).astype(jnp.int32)
    tab = jnp.pad(packed.reshape(-1), (0, _TAB_PAD - packed.size))
    ids_native = (dna.T.reshape(seqlen // _SB, _SB, nbatch // _BLK, _BLK)
                  .transpose(0, 2, 1, 3))
    y = _build(nbatch, seqlen)(ids_native, tab)
    return y.transpose(1, 3, 0, 2).reshape(nbatch, seqlen, 1, 4)


# speedup vs baseline: 1.2730x; 1.0060x over previous
import functools

import jax
import jax.numpy as jnp
from jax import lax
from jax.experimental import pallas as pl
from jax.experimental.pallas import tpu as pltpu
from jax.experimental.pallas import tpu_sc as plsc

_NC, _NS, _L = 2, 16, 16
_NW = _NC * _NS
_TAB_PAD = 352
_SB = 8
_BLK = 128
_NB = 4


@functools.lru_cache(maxsize=None)
def _build(nbatch, seqlen):
    nblk_b = nbatch // _BLK
    nblk_s = seqlen // _SB
    chunks_s = nblk_b // _NB
    n_chunks = nblk_s * chunks_s
    per_w = n_chunks // _NW
    assert per_w * _NW == n_chunks and per_w >= 3 and per_w % 2 == 1
    assert nblk_b * _BLK == nbatch and nblk_s * _SB == seqlen
    mesh = plsc.VectorSubcoreMesh(core_axis_name="c", subcore_axis_name="s")

    @functools.partial(
        pl.kernel,
        out_type=jax.ShapeDtypeStruct((seqlen, nblk_b, 4, _BLK), jnp.float32),
        mesh=mesh,
        scratch_types=[
            pltpu.VMEM((_TAB_PAD,), jnp.int32),
            pltpu.VMEM((2, _NB, _SB, _BLK), jnp.int32),
            pltpu.VMEM((2, _SB, _NB, 4, _BLK), jnp.float32),
            pltpu.SemaphoreType.DMA,
            pltpu.SemaphoreType.DMA,
            pltpu.SemaphoreType.DMA,
            pltpu.SemaphoreType.DMA,
        ],
        compiler_params=pltpu.CompilerParams(
            needs_layout_passes=False, use_tc_tiling_on_sc=False),
    )
    def k(ids_hbm, tab_hbm, out_hbm, tab_v, ids_v, out_v,
          isem0, isem1, osem0, osem1):
        isems = (isem0, isem1)
        osems = (osem0, osem1)
        wid = lax.axis_index("s") * _NC + lax.axis_index("c")
        k0 = wid * per_w
        pltpu.sync_copy(tab_hbm, tab_v)

        def ids_src(t):
            kk = k0 + t
            return ids_hbm.at[kk // chunks_s, pl.ds((kk % chunks_s) * _NB, _NB)]

        def out_dst(t):
            kk = k0 + t
            return out_hbm.at[pl.ds((kk // chunks_s) * _SB, _SB),
                              pl.ds((kk % chunks_s) * _NB, _NB)]

        def start_ids(t, buf):
            pltpu.async_copy(ids_src(t), ids_v.at[buf], isems[buf])

        def wait_ids(t, buf):
            pltpu.make_async_copy(ids_src(t), ids_v.at[buf],
                                  isems[buf]).wait()

        def start_out(t, buf):
            pltpu.async_copy(out_v.at[buf], out_dst(t), osems[buf])

        def wait_out(t, buf):
            pltpu.make_async_copy(out_v.at[buf], out_dst(t),
                                  osems[buf]).wait()

        himask = jnp.full((_L,), -65536, jnp.int32)

        def compute(buf):
            @plsc.parallel_loop(0, _NB * _SB, unroll=1)
            def q_body(q):
                bb = q // _SB
                s_lo = q % _SB
                for g in range(_BLK // _L):
                    ids16 = ids_v[buf, bb, s_lo, pl.ds(g * _L, _L)]
                    ids2 = ids16 * 2
                    for h in range(2):
                        w = plsc.load_gather(tab_v, [ids2 + h])
                        lo = plsc.bitcast(w << 16, jnp.float32)
                        hi = plsc.bitcast(w & himask, jnp.float32)
                        out_v[buf, s_lo, bb, 2 * h, pl.ds(g * _L, _L)] = lo
                        out_v[buf, s_lo, bb, 2 * h + 1, pl.ds(g * _L, _L)] = hi

        start_ids(0, 0)
        start_ids(1, 1)
        wait_ids(0, 0)
        compute(0)
        start_out(0, 0)
        start_ids(2, 0)
        wait_ids(1, 1)
        compute(1)
        start_out(1, 1)
        start_ids(3, 1)

        def loop_body(i, carry):
            t_even = 2 * i
            wait_out(t_even - 2, 0)
            wait_ids(t_even, 0)
            compute(0)
            start_out(t_even, 0)
            start_ids(t_even + 2, 0)
            t_odd = 2 * i + 1
            wait_out(t_odd - 2, 1)
            wait_ids(t_odd, 1)
            compute(1)
            start_out(t_odd, 1)

            @pl.when(t_odd + 2 < per_w)
            def _():
                start_ids(t_odd + 2, 1)
            return carry

        lax.fori_loop(1, (per_w - 1) // 2, loop_body, 0)
        t_last = per_w - 1
        wait_out(t_last - 2, 0)
        wait_ids(t_last, 0)
        compute(0)
        start_out(t_last, 0)
        wait_out(t_last - 1, 1)
        wait_out(t_last, 0)

    return k


def kernel(dna, embedding_table):
    nbatch, seqlen = dna.shape
    u16 = lax.bitcast_convert_type(
        embedding_table.astype(jnp.bfloat16), jnp.uint16).astype(jnp.uint32)
    packed = lax.bitcast_convert_type(
        u16[:, 0::2] | (u16[:, 1::2] << 16), jnp.int32)
    tab = jnp.pad(packed.reshape(-1), (0, _TAB_PAD - packed.size))
    ids_native = (dna.T.reshape(seqlen // _SB, _SB, nbatch // _BLK, _BLK)
                  .transpose(0, 2, 1, 3))
    y = _build(nbatch, seqlen)(ids_native, tab)
    return y.transpose(1, 3, 0, 2).reshape(nbatch, seqlen, 1, 4)
